# baseline (speedup 1.0000x reference)
import functools

import jax
import jax.numpy as jnp
from jax import lax
from jax.experimental import pallas as pl
from jax.experimental.pallas import tpu as pltpu
from jax.experimental.pallas import tpu_sc as plsc

_D = 32
_B = 16384
_NC = 2
_NS = 16
_NW = _NC * _NS
_BPW = _B // _NW

_mesh = plsc.VectorSubcoreMesh(core_axis_name="c", subcore_axis_name="s")


@functools.partial(
    pl.kernel,
    mesh=_mesh,
    compiler_params=pltpu.CompilerParams(use_tc_tiling_on_sc=False),
    out_type=jax.ShapeDtypeStruct((_B, _D), jnp.float32),
    scratch_types=[
        pltpu.VMEM((_BPW,), jnp.int32),
        pltpu.VMEM((_BPW, _D), jnp.float32),
        pltpu.SemaphoreType.DMA,
    ],
)
def _gather_kernel(table_hbm, idx_hbm, out_hbm, idx_v, rows_v, sem):
    wid = lax.axis_index("s") * _NC + lax.axis_index("c")
    base = wid * _BPW
    pltpu.sync_copy(idx_hbm.at[pl.ds(base, _BPW)], idx_v)
    pltpu.async_copy(table_hbm.at[idx_v], rows_v, sem).wait()
    pltpu.sync_copy(rows_v, out_hbm.at[pl.ds(base, _BPW)])


def kernel(titles, table):
    return _gather_kernel(table, titles.astype(jnp.int32))


# speedup vs baseline: 1.4006x; 1.4006x over previous
import functools

import jax
import jax.numpy as jnp
from jax import lax
from jax.experimental import pallas as pl
from jax.experimental.pallas import tpu as pltpu
from jax.experimental.pallas import tpu_sc as plsc

_D = 32
_B = 16384
_NC = 2
_NS = 16
_NW = _NC * _NS
_BPW = _B // _NW

_mesh = plsc.VectorSubcoreMesh(core_axis_name="c", subcore_axis_name="s")


@functools.partial(
    pl.kernel,
    mesh=_mesh,
    out_type=jax.ShapeDtypeStruct((_B, _D), jnp.float32),
    scratch_types=[
        pltpu.VMEM((_BPW,), jnp.int32),
        pltpu.VMEM((_BPW, _D), jnp.float32),
        pltpu.SemaphoreType.DMA,
    ],
)
def _gather_kernel(table_hbm, idx_hbm, out_hbm, idx_v, rows_v, sem):
    wid = lax.axis_index("s") * _NC + lax.axis_index("c")
    base = wid * _BPW
    pltpu.sync_copy(idx_hbm.at[pl.ds(base, _BPW)], idx_v)

    def fire(g, carry):
        vec = idx_v[pl.ds(g * 16, 16)]
        for j in range(16):
            r = vec[j]
            pltpu.async_copy(
                table_hbm.at[pl.ds(r, 1), :],
                rows_v.at[pl.ds(g * 16 + j, 1), :],
                sem,
            )
        return carry

    lax.fori_loop(0, _BPW // 16, fire, 0)
    pltpu.make_async_copy(table_hbm.at[pl.ds(0, _BPW), :], rows_v, sem).wait()
    pltpu.sync_copy(rows_v, out_hbm.at[pl.ds(base, _BPW)])


def kernel(titles, table):
    return _gather_kernel(table, titles.astype(jnp.int32))


# speedup vs baseline: 1.4038x; 1.0023x over previous
import functools

import jax
import jax.numpy as jnp
from jax import lax
from jax.experimental import pallas as pl
from jax.experimental.pallas import tpu as pltpu
from jax.experimental.pallas import tpu_sc as plsc

_D = 32
_B = 16384
_NC = 2
_NS = 16
_NW = _NC * _NS
_BPW = _B // _NW

_mesh = plsc.VectorSubcoreMesh(core_axis_name="c", subcore_axis_name="s")


@functools.partial(
    pl.kernel,
    mesh=_mesh,
    compiler_params=pltpu.CompilerParams(skip_device_barrier=True),
    out_type=jax.ShapeDtypeStruct((_B, _D), jnp.float32),
    scratch_types=[
        pltpu.VMEM((_BPW,), jnp.int32),
        pltpu.VMEM((_BPW, _D), jnp.float32),
        pltpu.SemaphoreType.DMA,
    ],
)
def _gather_kernel(table_hbm, idx_hbm, out_hbm, idx_v, rows_v, sem):
    wid = lax.axis_index("s") * _NC + lax.axis_index("c")
    base = wid * _BPW
    pltpu.sync_copy(idx_hbm.at[pl.ds(base, _BPW)], idx_v)

    def fire(g, carry):
        vec = idx_v[pl.ds(g * 16, 16)]
        for j in range(16):
            r = vec[j]
            pltpu.async_copy(
                table_hbm.at[pl.ds(r, 1), :],
                rows_v.at[pl.ds(g * 16 + j, 1), :],
                sem,
            )
        return carry

    lax.fori_loop(0, _BPW // 16, fire, 0)
    pltpu.make_async_copy(table_hbm.at[pl.ds(0, _BPW), :], rows_v, sem).wait()
    pltpu.sync_copy(rows_v, out_hbm.at[pl.ds(base, _BPW)])


def kernel(titles, table):
    return _gather_kernel(table, titles.astype(jnp.int32))


# speedup vs baseline: 2.3597x; 1.6809x over previous
import functools

import jax
import jax.numpy as jnp
from jax import lax
from jax.experimental import pallas as pl
from jax.experimental.pallas import tpu as pltpu
from jax.experimental.pallas import tpu_sc as plsc

_D = 32
_B = 16384
_V = 100001
_NC = 2
_H = _B // 2

_mesh = plsc.VectorSubcoreMesh(core_axis_name="c", subcore_axis_name="s")


@functools.partial(
    pl.kernel,
    mesh=_mesh,
    compiler_params=pltpu.CompilerParams(needs_layout_passes=False),
    out_type=jax.ShapeDtypeStruct((_D, _B), jnp.float32),
    scratch_types=[
        pltpu.VMEM((1, _V), jnp.float32),
        pltpu.VMEM((_H,), jnp.int32),
        pltpu.VMEM((1, _H), jnp.float32),
    ],
)
def _gather_kernel(tbl_hbm, idx_hbm, out_hbm, row_v, idx_v, orow_v):
    d = lax.axis_index("s") * _NC + lax.axis_index("c")
    pltpu.sync_copy(tbl_hbm.at[pl.ds(d, 1), :], row_v)

    def half(h):
        pltpu.sync_copy(idx_hbm.at[pl.ds(h * _H, _H)], idx_v)

        def grp(g, carry):
            vec = idx_v[pl.ds(g * 16, 16)]
            vals = plsc.load_gather(row_v, [vec * 0, vec])
            orow_v[0, pl.ds(g * 16, 16)] = vals
            return carry

        lax.fori_loop(0, _H // 16, grp, 0)
        pltpu.sync_copy(orow_v, out_hbm.at[pl.ds(d, 1), pl.ds(h * _H, _H)])

    half(0)
    half(1)


def kernel(titles, table):
    out_t = _gather_kernel(table.T, titles.astype(jnp.int32))
    return out_t.T


# speedup vs baseline: 2.5714x; 1.0897x over previous
import functools

import jax
import jax.numpy as jnp
from jax import lax
from jax.experimental import pallas as pl
from jax.experimental.pallas import tpu as pltpu
from jax.experimental.pallas import tpu_sc as plsc

_D = 32
_B = 16384
_V = 100001
_NC = 2
_H = _B // 2

_mesh = plsc.VectorSubcoreMesh(core_axis_name="c", subcore_axis_name="s")


@functools.partial(
    pl.kernel,
    mesh=_mesh,
    compiler_params=pltpu.CompilerParams(needs_layout_passes=False),
    out_type=jax.ShapeDtypeStruct((_D, _B), jnp.float32),
    scratch_types=[
        pltpu.VMEM((1, _V), jnp.float32),
        pltpu.VMEM((_H,), jnp.int32),
        pltpu.VMEM((_H,), jnp.int32),
        pltpu.VMEM((1, _H), jnp.float32),
        pltpu.SemaphoreType.DMA,
        pltpu.SemaphoreType.DMA,
    ],
)
def _gather_kernel(tbl_hbm, idx_hbm, out_hbm, row_v, idx0_v, idx1_v, orow_v,
                   rsem, isem):
    d = lax.axis_index("s") * _NC + lax.axis_index("c")
    row_cp = pltpu.async_copy(tbl_hbm.at[pl.ds(d, 1), :], row_v, rsem)
    idx0_cp = pltpu.async_copy(idx_hbm.at[pl.ds(0, _H)], idx0_v, isem)
    idx1_cp = pltpu.async_copy(idx_hbm.at[pl.ds(_H, _H)], idx1_v, isem)
    row_cp.wait()

    def half(h, idx_v):
        def grp(g, carry):
            for u in range(4):
                vec = idx_v[pl.ds((g * 4 + u) * 16, 16)]
                vals = plsc.load_gather(row_v, [vec * 0, vec])
                orow_v[0, pl.ds((g * 4 + u) * 16, 16)] = vals
            return carry

        lax.fori_loop(0, _H // 64, grp, 0)
        pltpu.sync_copy(orow_v, out_hbm.at[pl.ds(d, 1), pl.ds(h * _H, _H)])

    idx0_cp.wait()
    half(0, idx0_v)
    idx1_cp.wait()
    half(1, idx1_v)


def kernel(titles, table):
    out_t = _gather_kernel(table.T, titles.astype(jnp.int32))
    return out_t.T


# speedup vs baseline: 2.5756x; 1.0017x over previous
import functools

import jax
import jax.numpy as jnp
from jax import lax
from jax.experimental import pallas as pl
from jax.experimental.pallas import tpu as pltpu
from jax.experimental.pallas import tpu_sc as plsc

_D = 32
_B = 16384
_V = 100001
_NC = 2
_H = _B // 2

_mesh = plsc.VectorSubcoreMesh(core_axis_name="c", subcore_axis_name="s")


@functools.partial(
    pl.kernel,
    mesh=_mesh,
    compiler_params=pltpu.CompilerParams(needs_layout_passes=False),
    out_type=jax.ShapeDtypeStruct((_D, _B), jnp.float32),
    scratch_types=[
        pltpu.VMEM((_V,), jnp.float32),
        pltpu.VMEM((_H,), jnp.int32),
        pltpu.VMEM((_H,), jnp.int32),
        pltpu.VMEM((_H,), jnp.float32),
        pltpu.SemaphoreType.DMA,
        pltpu.SemaphoreType.DMA,
    ],
)
def _gather_kernel(tbl_hbm, idx_hbm, out_hbm, row_v, idx0_v, idx1_v, orow_v,
                   rsem, isem):
    d = lax.axis_index("s") * _NC + lax.axis_index("c")
    row_cp = pltpu.async_copy(tbl_hbm.at[d], row_v, rsem)
    idx0_cp = pltpu.async_copy(idx_hbm.at[pl.ds(0, _H)], idx0_v, isem)
    idx1_cp = pltpu.async_copy(idx_hbm.at[pl.ds(_H, _H)], idx1_v, isem)
    row_cp.wait()

    def half(h, idx_v):
        def grp(g, carry):
            for u in range(4):
                vec = idx_v[pl.ds((g * 4 + u) * 16, 16)]
                vals = plsc.load_gather(row_v, [vec])
                orow_v[pl.ds((g * 4 + u) * 16, 16)] = vals
            return carry

        lax.fori_loop(0, _H // 64, grp, 0)
        pltpu.sync_copy(orow_v, out_hbm.at[d, pl.ds(h * _H, _H)])

    idx0_cp.wait()
    half(0, idx0_v)
    idx1_cp.wait()
    half(1, idx1_v)


def kernel(titles, table):
    out_t = _gather_kernel(table.T, titles.astype(jnp.int32))
    return out_t.T
